# baseline (speedup 1.0000x reference)
import functools

import jax
import jax.numpy as jnp
from jax.experimental import pallas as pl
from jax.experimental.pallas import tpu as pltpu
from jax.experimental.pallas import tpu_sc as plsc

N2 = 16
NF = 7
W = 128
K = 8


def _sc_gather(tab3, idx3, b, l):
    mesh = plsc.VectorSubcoreMesh(core_axis_name="c", subcore_axis_name="s")
    out = jax.ShapeDtypeStruct((b, l, NF * N2), jnp.float32)
    bblocks = b // (K * W)

    @functools.partial(
        pl.kernel, out_type=(out, out), mesh=mesh,
        scratch_types=[pltpu.SemaphoreType.DMA],
        compiler_params=pltpu.CompilerParams(use_tc_tiling_on_sc=False,
                                             needs_layout_passes=False))
    def run(tabs_hbm, idx_hbm, ore_hbm, oim_hbm, sem):
        re_refs = [tabs_hbm.at[ff] for ff in range(NF)]
        im_refs = [tabs_hbm.at[NF + ff] for ff in range(NF)]

        def body(idxs, idx_v, ore_v, oim_v):
            _, f = idxs
            for ff in range(NF):
                @pl.when(f == ff)
                def _(ff=ff):
                    for k in range(K):
                        rows_ref = idx_v.at[0, k]
                        dst = pl.ds(k * W, W)
                        pltpu.async_copy(re_refs[ff].at[rows_ref],
                                         ore_v.at[dst, 0, :], sem)
                        pltpu.async_copy(im_refs[ff].at[rows_ref],
                                         oim_v.at[dst, 0, :], sem)
            for k in range(K):
                rows_ref = idx_v.at[0, k]
                dst = pl.ds(k * W, W)
                pltpu.make_async_copy(re_refs[0].at[rows_ref],
                                      ore_v.at[dst, 0, :], sem).wait()
                pltpu.make_async_copy(im_refs[0].at[rows_ref],
                                      oim_v.at[dst, 0, :], sem).wait()

        out_spec = pl.BlockSpec(
            (K * W, 1, N2),
            lambda i, f: (i % bblocks, i // bblocks, f))
        pltpu.emit_pipeline(
            body,
            grid=(b * l // (K * W), NF),
            in_specs=[pl.BlockSpec((1, K, W), lambda i, f: (f, i, 0))],
            out_specs=[out_spec, out_spec],
            core_axis_name=("c", "s"),
            dimension_semantics=(pltpu.PARALLEL, pltpu.PARALLEL),
            _explicit_indices=True,
        )(idx_hbm, ore_hbm, oim_hbm)

    return run(tab3, idx3)


def kernel(data, yr_re, yr_im, mt_re, mt_im, x_re, x_im, y_re, y_im,
           m_re, m_im, d_re, d_im, t_re, t_im):
    b, l, _ = data.shape
    tab3 = jnp.stack([yr_re, mt_re, x_re, y_re, m_re, d_re, t_re,
                      yr_im, mt_im, x_im, y_im, m_im, d_im, t_im])
    idx3 = jnp.transpose(data, (2, 1, 0)).reshape(NF, b * l // W, W)
    return _sc_gather(tab3, idx3, b, l)


# speedup vs baseline: 1.7083x; 1.7083x over previous
import functools

import jax
import jax.numpy as jnp
from jax.experimental import pallas as pl
from jax.experimental.pallas import tpu as pltpu
from jax.experimental.pallas import tpu_sc as plsc

N2 = 16
NF = 7
W = 128
K = 8


def _sc_gather(tabs, idx3, b, l, name):
    mesh = plsc.VectorSubcoreMesh(core_axis_name="c", subcore_axis_name="s")
    out = jax.ShapeDtypeStruct((b, l, NF * N2), jnp.float32)
    bblocks = b // (K * W)

    @functools.partial(
        pl.kernel, out_type=out, mesh=mesh, name=name,
        scratch_types=[pltpu.SemaphoreType.DMA],
        compiler_params=pltpu.CompilerParams(use_tc_tiling_on_sc=False,
                                             needs_layout_passes=False))
    def run(*refs):
        tab_refs = refs[:NF]
        idx_hbm, o_hbm, sem = refs[NF:]

        def body(idxs, idx_v, o_v):
            _, f = idxs
            for ff in range(NF):
                @pl.when(f == ff)
                def _(ff=ff):
                    for k in range(K):
                        pltpu.async_copy(tab_refs[ff].at[idx_v.at[0, k]],
                                         o_v.at[pl.ds(k * W, W), 0, :], sem)
            for k in range(K):
                pltpu.make_async_copy(tab_refs[0].at[idx_v.at[0, k]],
                                      o_v.at[pl.ds(k * W, W), 0, :], sem).wait()

        out_spec = pl.BlockSpec(
            (K * W, 1, N2),
            lambda i, f: (i % bblocks, i // bblocks, f))
        pltpu.emit_pipeline(
            body,
            grid=(b * l // (K * W), NF),
            in_specs=[pl.BlockSpec((1, K, W), lambda i, f: (f, i, 0))],
            out_specs=[out_spec],
            core_axis_name=("c", "s"),
            dimension_semantics=(pltpu.PARALLEL, pltpu.PARALLEL),
            _explicit_indices=True,
        )(idx_hbm, o_hbm)

    return run(*tabs, idx3)


def kernel(data, yr_re, yr_im, mt_re, mt_im, x_re, x_im, y_re, y_im,
           m_re, m_im, d_re, d_im, t_re, t_im):
    b, l, _ = data.shape
    idx3 = jnp.transpose(data, (2, 1, 0)).reshape(NF, b * l // W, W)
    ore = _sc_gather((yr_re, mt_re, x_re, y_re, m_re, d_re, t_re),
                     idx3, b, l, "gather_re")
    oim = _sc_gather((yr_im, mt_im, x_im, y_im, m_im, d_im, t_im),
                     idx3, b, l, "gather_im")
    return ore, oim
